# baseline (speedup 1.0000x reference)
import jax
import jax.numpy as jnp
from jax.experimental import pallas as pl
from jax.experimental.pallas import tpu as pltpu

N = 4096
HDIM = 128
INDIM = 64
BETA = 0.05
TM = 256
NS = N // TM
E5 = jnp.float8_e5m2
BF = jnp.bfloat16
F32 = jnp.float32


def _nt_dot(lhs, rhs):
    return jax.lax.dot_general(lhs, rhs, (((1,), (1,)), ((), ())),
                               preferred_element_type=F32)


def _tn_dot(lhs, rhs):
    return jax.lax.dot_general(lhs, rhs, (((0,), (0,)), ((), ())),
                               preferred_element_type=F32)


def _body(ct_ref, at_ref, h8t_ref, ht_ref, xt_ref,
          wm_ref, bm_ref, wz_ref, bz_ref, wg_ref, bg_ref,
          outt_ref, a8_s, c8_s, h1at_s, h1ct_s, h1a8t_s, h1c8t_s):
    p = pl.program_id(0)
    i = pl.program_id(1)
    rows = pl.ds(i * TM, TM)

    def hop1(src_ref, s8, s1, s18):
        s = src_ref[...].astype(E5)
        s8[rows, :] = s
        h1t = (BETA * ht_ref[:, rows]
               + (1.0 - BETA) * _nt_dot(h8t_ref[...], s))
        s1[:, rows] = h1t.astype(BF)
        s18[:, rows] = h1t.astype(E5)

    @pl.when(p == 0)
    def _hop1():
        hop1(ct_ref, c8_s, h1ct_s, h1c8t_s)
        hop1(at_ref, a8_s, h1at_s, h1a8t_s)

    @pl.when(p == 1)
    def _hop2_epilogue():
        htw = ht_ref[:, rows]
        mix = BETA * htw
        h2ct = mix + (1.0 - BETA) * _nt_dot(h1c8t_s[...], c8_s[rows, :])
        h2at = mix + (1.0 - BETA) * _nt_dot(h1a8t_s[...], a8_s[rows, :])
        h_cat_t = jnp.concatenate(
            [htw.astype(BF), h1at_s[:, rows], h2at.astype(BF),
             h1ct_s[:, rows], h2ct.astype(BF)], axis=0)
        h_g_t = _tn_dot(wm_ref[...], h_cat_t) + bm_ref[...]
        inp_t = jnp.concatenate([h_g_t.astype(BF), xt_ref[...]], axis=0)
        z = jax.nn.sigmoid(_tn_dot(wz_ref[...], inp_t) + bz_ref[...])
        g = jnp.tanh(_tn_dot(wg_ref[...], inp_t) + bg_ref[...])
        outt_ref[...] = z * htw + (1.0 - z) * g


@jax.jit
def kernel(t, H_in, X_in, A, C, W_mlp, b_mlp, W_z, b_z, W_g, b_g):
    del t
    grid = (2, NS)
    def stream():
        return pl.BlockSpec(
            (TM, N), lambda p, i: (jnp.where(p == 0, i, NS - 1), 0))

    def full(shape):
        return pl.BlockSpec(shape, lambda p, i: tuple(0 for _ in shape))

    H_t = H_in.T
    H8_t = H_t.astype(E5)
    X_t = X_in.T.astype(BF)
    out_t = pl.pallas_call(
        _body,
        grid=grid,
        in_specs=[stream(), stream(),
                  full((HDIM, N)), full((HDIM, N)),
                  pl.BlockSpec((INDIM, TM),
                               lambda p, i: (0, jnp.where(p == 1, i, 0))),
                  full((5 * HDIM, HDIM)), full((HDIM, 1)),
                  full((HDIM + INDIM, HDIM)), full((HDIM, 1)),
                  full((HDIM + INDIM, HDIM)), full((HDIM, 1))],
        out_specs=pl.BlockSpec((HDIM, TM),
                               lambda p, i: (0, jnp.where(p == 1, i, 0))),
        out_shape=jax.ShapeDtypeStruct((HDIM, N), F32),
        scratch_shapes=[
            pltpu.VMEM((N, N), E5),
            pltpu.VMEM((N, N), E5),
            pltpu.VMEM((HDIM, N), BF),
            pltpu.VMEM((HDIM, N), BF),
            pltpu.VMEM((HDIM, N), E5),
            pltpu.VMEM((HDIM, N), E5),
        ],
        compiler_params=pltpu.CompilerParams(
            dimension_semantics=("arbitrary", "arbitrary"),
            vmem_limit_bytes=100 * 1024 * 1024),
    )(C, A, H8_t, H_t, X_t,
      W_mlp.astype(BF), b_mlp.reshape(HDIM, 1),
      W_z.astype(BF), b_z.reshape(HDIM, 1),
      W_g.astype(BF), b_g.reshape(HDIM, 1))
    return out_t.T


# speedup vs baseline: 1.1245x; 1.1245x over previous
import jax
import jax.numpy as jnp
from jax.experimental import pallas as pl
from jax.experimental.pallas import tpu as pltpu

N = 4096
HDIM = 128
INDIM = 64
BETA = 0.05
TM = 256
SA = 4096.0
E4 = jnp.float8_e4m3fn
BF = jnp.bfloat16


def _body(a_ref, c_ref, h8_ref, ht_ref, xt_ref,
          wm_ref, bm_ref, wz_ref, bz_ref, wg_ref, bg_ref,
          out_ref, a8_s, c8_s, h1a_s, h1c_s, h1a8_s, h1c8_s):
    p = pl.program_id(0)
    i = pl.program_id(1)
    rows = pl.ds(i * TM, TM)
    ht = ht_ref[...]
    mix = BETA * ht

    @pl.when(p == 0)
    def _hop1():
        a_bf = (SA * a_ref[...]).astype(BF)
        c_bf = (SA * c_ref[...]).astype(BF)
        a8_s[rows, :] = a_bf.astype(E4)
        c8_s[rows, :] = c_bf.astype(E4)
        h8 = h8_ref[...]
        h1a = mix + jnp.dot(a_bf, h8, preferred_element_type=jnp.float32)
        h1c = mix + jnp.dot(c_bf, h8, preferred_element_type=jnp.float32)
        h1a_s[rows, :] = h1a.astype(BF)
        h1c_s[rows, :] = h1c.astype(BF)
        h1a8_s[rows, :] = h1a.astype(E4)
        h1c8_s[rows, :] = h1c.astype(E4)

    @pl.when(p == 1)
    def _hop2_epilogue():
        h2a = mix + ((1.0 - BETA) / SA) * jnp.dot(
            a8_s[rows, :], h1a8_s[...], preferred_element_type=jnp.float32)
        h2c = mix + ((1.0 - BETA) / SA) * jnp.dot(
            c8_s[rows, :], h1c8_s[...], preferred_element_type=jnp.float32)
        h_cat = jnp.concatenate(
            [ht.astype(BF), h1a_s[rows, :], h2a.astype(BF),
             h1c_s[rows, :], h2c.astype(BF)], axis=1)
        h_g = jnp.dot(h_cat, wm_ref[...],
                      preferred_element_type=jnp.float32) + bm_ref[...]
        inp = jnp.concatenate([h_g.astype(BF), xt_ref[...]], axis=1)
        z = jax.nn.sigmoid(
            jnp.dot(inp, wz_ref[...],
                    preferred_element_type=jnp.float32) + bz_ref[...])
        g = jnp.tanh(
            jnp.dot(inp, wg_ref[...],
                    preferred_element_type=jnp.float32) + bg_ref[...])
        out_ref[...] = z * ht + (1.0 - z) * g


@jax.jit
def kernel(t, H_in, X_in, A, C, W_mlp, b_mlp, W_z, b_z, W_g, b_g):
    del t
    grid = (2, N // TM)
    ac_spec = pl.BlockSpec((TM, N), lambda p, i: (i * (1 - p), 0))
    h_tile = pl.BlockSpec((TM, HDIM), lambda p, i: (i, 0))

    def full(shape):
        return pl.BlockSpec(shape, lambda p, i: tuple(0 for _ in shape))

    H8 = (H_in * ((1.0 - BETA) / SA)).astype(BF)
    X_bf = X_in.astype(BF)
    Wm_bf = W_mlp.astype(BF)
    Wz_bf = W_z.astype(BF)
    Wg_bf = W_g.astype(BF)
    bm2 = b_mlp.reshape(1, HDIM)
    bz2 = b_z.reshape(1, HDIM)
    bg2 = b_g.reshape(1, HDIM)

    out = pl.pallas_call(
        _body,
        grid=grid,
        in_specs=[ac_spec, ac_spec, full((N, HDIM)), h_tile,
                  pl.BlockSpec((TM, INDIM), lambda p, i: (i, 0)),
                  full((5 * HDIM, HDIM)), full((1, HDIM)),
                  full((HDIM + INDIM, HDIM)), full((1, HDIM)),
                  full((HDIM + INDIM, HDIM)), full((1, HDIM))],
        out_specs=pl.BlockSpec((TM, HDIM), lambda p, i: (i * p, 0)),
        out_shape=jax.ShapeDtypeStruct((N, HDIM), jnp.float32),
        scratch_shapes=[
            pltpu.VMEM((N, N), E4),
            pltpu.VMEM((N, N), E4),
            pltpu.VMEM((N, HDIM), BF),
            pltpu.VMEM((N, HDIM), BF),
            pltpu.VMEM((N, HDIM), E4),
            pltpu.VMEM((N, HDIM), E4),
        ],
        compiler_params=pltpu.CompilerParams(
            dimension_semantics=("arbitrary", "arbitrary"),
            vmem_limit_bytes=100 * 1024 * 1024),
    )(A, C, H8, H_in, X_bf, Wm_bf, bm2, Wz_bf, bz2, Wg_bf, bg2)
    return out


# speedup vs baseline: 1.1486x; 1.0214x over previous
import jax
import jax.numpy as jnp
from jax.experimental import pallas as pl
from jax.experimental.pallas import tpu as pltpu

N = 4096
HDIM = 128
INDIM = 64
BETA = 0.05
TM = 256
E5 = jnp.float8_e5m2
BF = jnp.bfloat16


def _body(a_ref, c_ref, h8_ref, ht_ref, xt_ref,
          wm_ref, bm_ref, wz_ref, bz_ref, wg_ref, bg_ref,
          out_ref, a8_s, c8_s, h1a_s, h1c_s, h1a8_s, h1c8_s):
    p = pl.program_id(0)
    i = pl.program_id(1)
    rows = pl.ds(i * TM, TM)
    ht = ht_ref[...]
    mix = BETA * ht

    @pl.when(p == 0)
    def _hop1():
        a_bf = a_ref[...].astype(BF)
        c_bf = c_ref[...].astype(BF)
        a8_s[rows, :] = a_bf.astype(E5)
        c8_s[rows, :] = c_bf.astype(E5)
        h8 = h8_ref[...]
        h1a = mix + jnp.dot(a_bf, h8, preferred_element_type=jnp.float32)
        h1c = mix + jnp.dot(c_bf, h8, preferred_element_type=jnp.float32)
        h1a_s[rows, :] = h1a.astype(BF)
        h1c_s[rows, :] = h1c.astype(BF)
        h1a8_s[rows, :] = h1a.astype(E5)
        h1c8_s[rows, :] = h1c.astype(E5)

    @pl.when(p == 1)
    def _hop2_epilogue():
        h2a = mix + (1.0 - BETA) * jnp.dot(
            a8_s[rows, :], h1a8_s[...], preferred_element_type=jnp.float32)
        h2c = mix + (1.0 - BETA) * jnp.dot(
            c8_s[rows, :], h1c8_s[...], preferred_element_type=jnp.float32)
        h_cat = jnp.concatenate(
            [ht.astype(BF), h1a_s[rows, :], h2a.astype(BF),
             h1c_s[rows, :], h2c.astype(BF)], axis=1)
        h_g = jnp.dot(h_cat, wm_ref[...],
                      preferred_element_type=jnp.float32) + bm_ref[...]
        inp = jnp.concatenate([h_g.astype(BF), xt_ref[...]], axis=1)
        z = jax.nn.sigmoid(
            jnp.dot(inp, wz_ref[...],
                    preferred_element_type=jnp.float32) + bz_ref[...])
        g = jnp.tanh(
            jnp.dot(inp, wg_ref[...],
                    preferred_element_type=jnp.float32) + bg_ref[...])
        out_ref[...] = z * ht + (1.0 - z) * g


@jax.jit
def kernel(t, H_in, X_in, A, C, W_mlp, b_mlp, W_z, b_z, W_g, b_g):
    del t
    grid = (2, N // TM)
    ac_spec = pl.BlockSpec((TM, N), lambda p, i: (i * (1 - p), 0))
    h_tile = pl.BlockSpec((TM, HDIM), lambda p, i: (i, 0))

    def full(shape):
        return pl.BlockSpec(shape, lambda p, i: tuple(0 for _ in shape))

    H8 = (H_in * (1.0 - BETA)).astype(BF)
    X_bf = X_in.astype(BF)
    Wm_bf = W_mlp.astype(BF)
    Wz_bf = W_z.astype(BF)
    Wg_bf = W_g.astype(BF)
    bm2 = b_mlp.reshape(1, HDIM)
    bz2 = b_z.reshape(1, HDIM)
    bg2 = b_g.reshape(1, HDIM)

    out = pl.pallas_call(
        _body,
        grid=grid,
        in_specs=[ac_spec, ac_spec, full((N, HDIM)), h_tile,
                  pl.BlockSpec((TM, INDIM), lambda p, i: (i, 0)),
                  full((5 * HDIM, HDIM)), full((1, HDIM)),
                  full((HDIM + INDIM, HDIM)), full((1, HDIM)),
                  full((HDIM + INDIM, HDIM)), full((1, HDIM))],
        out_specs=pl.BlockSpec((TM, HDIM), lambda p, i: (i * p, 0)),
        out_shape=jax.ShapeDtypeStruct((N, HDIM), jnp.float32),
        scratch_shapes=[
            pltpu.VMEM((N, N), E5),
            pltpu.VMEM((N, N), E5),
            pltpu.VMEM((N, HDIM), BF),
            pltpu.VMEM((N, HDIM), BF),
            pltpu.VMEM((N, HDIM), E5),
            pltpu.VMEM((N, HDIM), E5),
        ],
        compiler_params=pltpu.CompilerParams(
            dimension_semantics=("arbitrary", "arbitrary"),
            vmem_limit_bytes=100 * 1024 * 1024),
    )(A, C, H8, H_in, X_bf, Wm_bf, bm2, Wz_bf, bz2, Wg_bf, bg2)
    return out
